# Initial kernel scaffold; baseline (speedup 1.0000x reference)
import jax
import jax.numpy as jnp
from jax.experimental import pallas as pl


def kernel(x, edge_index, batch, params):
    raise NotImplementedError("write your pallas kernel here")


# speedup vs baseline: 2.0121x; 2.0121x over previous
import functools

import numpy as np
import jax
import jax.numpy as jnp
from jax import lax
from jax.experimental import pallas as pl
from jax.experimental.pallas import tpu as pltpu
from jax.experimental.pallas import tpu_sc as plsc

_N = 10000
_E = 320000
_F = 128
_G = 64
_NSPL = 8
_NC = 2
_NS = 16
_NW = _NC * _NS
_CHUNK = 128
_EDGES_PER_TILE = 10240
_EP = _NW * _EDGES_PER_TILE
_CHUNKS_PER_TILE = _EDGES_PER_TILE // _CHUNK
_IDX_ROWS = _EP // _CHUNK
_ACC_ROWS = 10240
_ZROWS = _ACC_ROWS // _NS
_B = 1000
_NBLK = _N // _B

_GRIDV = [float(v) for v in
          (np.arange(-3, 9, dtype=np.float32) * np.float32(2.0 / 5)
           - np.float32(1.0))]


def _sc_segment_sum(h, src2d, dst2d, zblk):
    mesh = plsc.VectorSubcoreMesh(core_axis_name="c", subcore_axis_name="s")

    @functools.partial(
        pl.kernel,
        out_type=jax.ShapeDtypeStruct((_NC, _ACC_ROWS, _F), jnp.float32),
        mesh=mesh,
        scratch_types=[
            pltpu.VMEM((_CHUNK,), jnp.int32),
            pltpu.VMEM((_CHUNK,), jnp.int32),
            pltpu.VMEM((_CHUNK, _F), jnp.float32),
            pltpu.VMEM_SHARED((_ACC_ROWS, _F), jnp.float32),
            pltpu.SemaphoreType.DMA,
        ],
    )
    def seg_kernel(h_hbm, src_hbm, dst_hbm, z_hbm, out_hbm,
                   sidx, didx, rows, acc, sem):
        c = lax.axis_index("c")
        s = lax.axis_index("s")
        pltpu.sync_copy(z_hbm, acc.at[pl.ds(s * _ZROWS, _ZROWS)])
        plsc.subcore_barrier()

        base = (c * _NS + s) * _CHUNKS_PER_TILE

        @pl.loop(0, _CHUNKS_PER_TILE)
        def _(j):
            pltpu.sync_copy(src_hbm.at[base + j], sidx)
            pltpu.sync_copy(dst_hbm.at[base + j], didx)
            pltpu.async_copy(h_hbm.at[sidx], rows, sem).wait()
            pltpu.sync_copy(rows, acc.at[didx], add=True)

        plsc.subcore_barrier()
        pltpu.sync_copy(acc.at[pl.ds(s * _ZROWS, _ZROWS)],
                        out_hbm.at[c, pl.ds(s * _ZROWS, _ZROWS)])

    return seg_kernel(h, src2d, dst2d, zblk)


def _dot(a, b):
    return lax.dot_general(a, b, (((1,), (0,)), ((), ())),
                           preferred_element_type=jnp.float32,
                           precision=lax.Precision.HIGHEST)


def _kan(x, bwT, w_ref):
    sig = 1.0 / (1.0 + jnp.exp(-x))
    out = _dot(x * sig, bwT)
    g = _GRIDV
    bases = [jnp.where((x >= g[j]) & (x < g[j + 1]), 1.0, 0.0)
             for j in range(11)]
    for k in range(1, 4):
        nxt = []
        for j in range(11 - k):
            left = (x - g[j]) * (1.0 / (g[j + k] - g[j]))
            right = (g[j + k + 1] - x) * (1.0 / (g[j + k + 1] - g[j + 1]))
            nxt.append(left * bases[j] + right * bases[j + 1])
        bases = nxt
    for k in range(_NSPL):
        out = out + _dot(bases[k], w_ref[k, :, :])
    return out


def _layer_compute(eps_ref, h_ref, a0_ref, a1_ref, bw1_ref, w1_ref,
                   bw2_ref, w2_ref):
    eps = eps_ref[0, 0]
    h2 = (1.0 + eps) * h_ref[...] + a0_ref[...] + a1_ref[...]
    t = _kan(h2, bw1_ref[...], w1_ref)
    return _kan(t, bw2_ref[...], w2_ref)


def _mid_body(eps_ref, h_ref, a0_ref, a1_ref, bw1_ref, w1_ref,
              bw2_ref, w2_ref, o_ref):
    t = _layer_compute(eps_ref, h_ref, a0_ref, a1_ref, bw1_ref, w1_ref,
                       bw2_ref, w2_ref)
    o_ref[...] = jnp.where(t >= 0, t, 0.01 * t)


def _last_body(eps_ref, h_ref, a0_ref, a1_ref, bw1_ref, w1_ref,
               bw2_ref, w2_ref, b_ref, pool_ref, cnt_ref):
    i = pl.program_id(0)

    @pl.when(i == 0)
    def _():
        pool_ref[...] = jnp.zeros_like(pool_ref)
        cnt_ref[...] = jnp.zeros_like(cnt_ref)

    t = _layer_compute(eps_ref, h_ref, a0_ref, a1_ref, bw1_ref, w1_ref,
                       bw2_ref, w2_ref)
    bvec = b_ref[0, 0, :]
    onehot = (bvec[:, None]
              == lax.broadcasted_iota(jnp.int32, (_B, _G), 1)
              ).astype(jnp.float32)
    pool_ref[...] += lax.dot_general(
        onehot, t, (((0,), (0,)), ((), ())),
        preferred_element_type=jnp.float32, precision=lax.Precision.HIGHEST)
    cnt_ref[...] += lax.dot_general(
        onehot, jnp.ones((_B, _F), jnp.float32), (((0,), (0,)), ((), ())),
        preferred_element_type=jnp.float32)

    @pl.when(i == _NBLK - 1)
    def _():
        pool_ref[...] = pool_ref[...] / jnp.maximum(cnt_ref[...], 1.0)


_EPS_SPEC = pl.BlockSpec((1, 1), lambda i: (0, 0))
_H_SPEC = pl.BlockSpec((_B, _F), lambda i: (i, 0))
_BW_SPEC = pl.BlockSpec((_F, _F), lambda i: (0, 0))
_W_SPEC = pl.BlockSpec((_NSPL, _F, _F), lambda i: (0, 0, 0))


def _tc_mid(eps11, h, a0, a1, bw1T, w1, bw2T, w2):
    return pl.pallas_call(
        _mid_body,
        grid=(_NBLK,),
        in_specs=[_EPS_SPEC, _H_SPEC, _H_SPEC, _H_SPEC,
                  _BW_SPEC, _W_SPEC, _BW_SPEC, _W_SPEC],
        out_specs=_H_SPEC,
        out_shape=jax.ShapeDtypeStruct((_N, _F), jnp.float32),
    )(eps11, h, a0, a1, bw1T, w1, bw2T, w2)


def _tc_last(eps11, h, a0, a1, bw1T, w1, bw2T, w2, batch3):
    return pl.pallas_call(
        _last_body,
        grid=(_NBLK,),
        in_specs=[_EPS_SPEC, _H_SPEC, _H_SPEC, _H_SPEC,
                  _BW_SPEC, _W_SPEC, _BW_SPEC, _W_SPEC,
                  pl.BlockSpec((1, 1, _B), lambda i: (i, 0, 0))],
        out_specs=pl.BlockSpec((_G, _F), lambda i: (0, 0)),
        out_shape=jax.ShapeDtypeStruct((_G, _F), jnp.float32),
        scratch_shapes=[pltpu.VMEM((_G, _F), jnp.float32)],
    )(eps11, h, a0, a1, bw1T, w1, bw2T, w2, batch3)


def kernel(x, edge_index, batch, params):
    src = edge_index[0].astype(jnp.int32)
    dst = edge_index[1].astype(jnp.int32)
    pad = _EP - _E
    src2d = jnp.concatenate(
        [src, jnp.zeros((pad,), jnp.int32)]).reshape(_IDX_ROWS, _CHUNK)
    dst2d = jnp.concatenate(
        [dst, jnp.full((pad,), _N, jnp.int32)]).reshape(_IDX_ROWS, _CHUNK)
    zblk = jnp.zeros((_ZROWS, _F), jnp.float32)
    batch3 = batch.astype(jnp.int32).reshape(_NBLK, 1, _B)

    layer_args = []
    for (eps, p0, p1) in params:
        layer_args.append((
            jnp.reshape(eps, (1, 1)).astype(jnp.float32),
            p0[0].T,
            jnp.transpose(p0[1] * p0[2][..., None], (2, 1, 0)),
            p1[0].T,
            jnp.transpose(p1[1] * p1[2][..., None], (2, 1, 0)),
        ))

    h = x
    pooled = None
    for li in range(3):
        eps11, bw1T, w1, bw2T, w2 = layer_args[li]
        agg = _sc_segment_sum(h, src2d, dst2d, zblk)
        if li < 2:
            h = _tc_mid(eps11, h, agg[0], agg[1], bw1T, w1, bw2T, w2)
        else:
            pooled = _tc_last(eps11, h, agg[0], agg[1], bw1T, w1, bw2T, w2,
                              batch3)
    return (pooled, 0)


# speedup vs baseline: 4.0907x; 2.0330x over previous
import functools

import numpy as np
import jax
import jax.numpy as jnp
from jax import lax
from jax.experimental import pallas as pl
from jax.experimental.pallas import tpu as pltpu
from jax.experimental.pallas import tpu_sc as plsc

_N = 10000
_E = 320000
_F = 128
_G = 64
_NSPL = 8
_NC = 2
_NS = 16
_NW = _NC * _NS
_CHUNK = 128
_EDGES_PER_TILE = 10240
_EP = _NW * _EDGES_PER_TILE
_CHUNKS_PER_TILE = _EDGES_PER_TILE // _CHUNK
_STAGE_CHUNKS = 40
_IDX_ROWS = _EP // _CHUNK
_ACC_ROWS = 10240
_ZROWS = _ACC_ROWS // _NS
_B = 1000
_NBLK = _N // _B

_GRIDV = [float(v) for v in
          (np.arange(-3, 9, dtype=np.float32) * np.float32(2.0 / 5)
           - np.float32(1.0))]


def _sc_segment_sum(h, src2d, dst2d, zblk):
    mesh = plsc.VectorSubcoreMesh(core_axis_name="c", subcore_axis_name="s")

    @functools.partial(
        pl.kernel,
        out_type=jax.ShapeDtypeStruct((_NC, _ACC_ROWS, _F), jnp.float32),
        mesh=mesh,
        scratch_types=[
            pltpu.VMEM((_STAGE_CHUNKS, _CHUNK), jnp.int32),
            pltpu.VMEM((_STAGE_CHUNKS, _CHUNK), jnp.int32),
            pltpu.VMEM((_CHUNK, _F), jnp.float32),
            pltpu.VMEM((_CHUNK, _F), jnp.float32),
            pltpu.VMEM_SHARED((_ACC_ROWS, _F), jnp.float32),
            pltpu.SemaphoreType.DMA,
            pltpu.SemaphoreType.DMA,
        ],
    )
    def seg_kernel(h_hbm, src_hbm, dst_hbm, z_hbm, out_hbm,
                   sidx, didx, rows_a, rows_b, acc, sem_a, sem_b):
        c = lax.axis_index("c")
        s = lax.axis_index("s")
        pltpu.sync_copy(z_hbm, acc.at[pl.ds(s * _ZROWS, _ZROWS)])
        row0 = (c * _NS + s) * _CHUNKS_PER_TILE
        plsc.subcore_barrier()

        for t in range(_CHUNKS_PER_TILE // _STAGE_CHUNKS):
            pltpu.sync_copy(
                src_hbm.at[pl.ds(row0 + t * _STAGE_CHUNKS, _STAGE_CHUNKS)],
                sidx)
            pltpu.sync_copy(
                dst_hbm.at[pl.ds(row0 + t * _STAGE_CHUNKS, _STAGE_CHUNKS)],
                didx)
            pltpu.async_copy(h_hbm.at[sidx.at[0]], rows_a, sem_a)

            @pl.loop(0, _STAGE_CHUNKS // 2)
            def _(k):
                j = k * 2
                pltpu.async_copy(h_hbm.at[sidx.at[j + 1]], rows_b, sem_b)
                pltpu.make_async_copy(h_hbm.at[sidx.at[0]], rows_a,
                                      sem_a).wait()
                pltpu.sync_copy(rows_a, acc.at[didx.at[j]], add=True)

                @pl.when(j + 2 < _STAGE_CHUNKS)
                def _():
                    pltpu.async_copy(h_hbm.at[sidx.at[j + 2]], rows_a, sem_a)

                pltpu.make_async_copy(h_hbm.at[sidx.at[0]], rows_b,
                                      sem_b).wait()
                pltpu.sync_copy(rows_b, acc.at[didx.at[j + 1]], add=True)

        plsc.subcore_barrier()
        pltpu.sync_copy(acc.at[pl.ds(s * _ZROWS, _ZROWS)],
                        out_hbm.at[c, pl.ds(s * _ZROWS, _ZROWS)])

    return seg_kernel(h, src2d, dst2d, zblk)


def _dot(a, b):
    return lax.dot_general(a, b, (((1,), (0,)), ((), ())),
                           preferred_element_type=jnp.float32,
                           precision=lax.Precision.HIGHEST)


def _kan(x, bwT, w_ref):
    sig = 1.0 / (1.0 + jnp.exp(-x))
    out = _dot(x * sig, bwT)
    g = _GRIDV
    bases = [jnp.where((x >= g[j]) & (x < g[j + 1]), 1.0, 0.0)
             for j in range(11)]
    for k in range(1, 4):
        nxt = []
        for j in range(11 - k):
            left = (x - g[j]) * (1.0 / (g[j + k] - g[j]))
            right = (g[j + k + 1] - x) * (1.0 / (g[j + k + 1] - g[j + 1]))
            nxt.append(left * bases[j] + right * bases[j + 1])
        bases = nxt
    for k in range(_NSPL):
        out = out + _dot(bases[k], w_ref[k, :, :])
    return out


def _layer_compute(eps_ref, h_ref, a0_ref, a1_ref, bw1_ref, w1_ref,
                   bw2_ref, w2_ref):
    eps = eps_ref[0, 0]
    h2 = (1.0 + eps) * h_ref[...] + a0_ref[...] + a1_ref[...]
    t = _kan(h2, bw1_ref[...], w1_ref)
    return _kan(t, bw2_ref[...], w2_ref)


def _mid_body(eps_ref, h_ref, a0_ref, a1_ref, bw1_ref, w1_ref,
              bw2_ref, w2_ref, o_ref):
    t = _layer_compute(eps_ref, h_ref, a0_ref, a1_ref, bw1_ref, w1_ref,
                       bw2_ref, w2_ref)
    o_ref[...] = jnp.where(t >= 0, t, 0.01 * t)


def _last_body(eps_ref, h_ref, a0_ref, a1_ref, bw1_ref, w1_ref,
               bw2_ref, w2_ref, b_ref, pool_ref, cnt_ref):
    i = pl.program_id(0)

    @pl.when(i == 0)
    def _():
        pool_ref[...] = jnp.zeros_like(pool_ref)
        cnt_ref[...] = jnp.zeros_like(cnt_ref)

    t = _layer_compute(eps_ref, h_ref, a0_ref, a1_ref, bw1_ref, w1_ref,
                       bw2_ref, w2_ref)
    bvec = b_ref[0, 0, :]
    onehot = (bvec[:, None]
              == lax.broadcasted_iota(jnp.int32, (_B, _G), 1)
              ).astype(jnp.float32)
    pool_ref[...] += lax.dot_general(
        onehot, t, (((0,), (0,)), ((), ())),
        preferred_element_type=jnp.float32, precision=lax.Precision.HIGHEST)
    cnt_ref[...] += lax.dot_general(
        onehot, jnp.ones((_B, _F), jnp.float32), (((0,), (0,)), ((), ())),
        preferred_element_type=jnp.float32)

    @pl.when(i == _NBLK - 1)
    def _():
        pool_ref[...] = pool_ref[...] / jnp.maximum(cnt_ref[...], 1.0)


_EPS_SPEC = pl.BlockSpec((1, 1), lambda i: (0, 0))
_H_SPEC = pl.BlockSpec((_B, _F), lambda i: (i, 0))
_BW_SPEC = pl.BlockSpec((_F, _F), lambda i: (0, 0))
_W_SPEC = pl.BlockSpec((_NSPL, _F, _F), lambda i: (0, 0, 0))


def _tc_mid(eps11, h, a0, a1, bw1T, w1, bw2T, w2):
    return pl.pallas_call(
        _mid_body,
        grid=(_NBLK,),
        in_specs=[_EPS_SPEC, _H_SPEC, _H_SPEC, _H_SPEC,
                  _BW_SPEC, _W_SPEC, _BW_SPEC, _W_SPEC],
        out_specs=_H_SPEC,
        out_shape=jax.ShapeDtypeStruct((_N, _F), jnp.float32),
    )(eps11, h, a0, a1, bw1T, w1, bw2T, w2)


def _tc_last(eps11, h, a0, a1, bw1T, w1, bw2T, w2, batch3):
    return pl.pallas_call(
        _last_body,
        grid=(_NBLK,),
        in_specs=[_EPS_SPEC, _H_SPEC, _H_SPEC, _H_SPEC,
                  _BW_SPEC, _W_SPEC, _BW_SPEC, _W_SPEC,
                  pl.BlockSpec((1, 1, _B), lambda i: (i, 0, 0))],
        out_specs=pl.BlockSpec((_G, _F), lambda i: (0, 0)),
        out_shape=jax.ShapeDtypeStruct((_G, _F), jnp.float32),
        scratch_shapes=[pltpu.VMEM((_G, _F), jnp.float32)],
    )(eps11, h, a0, a1, bw1T, w1, bw2T, w2, batch3)


def kernel(x, edge_index, batch, params):
    src = edge_index[0].astype(jnp.int32)
    dst = edge_index[1].astype(jnp.int32)
    pad = _EP - _E
    pad_ar = np.arange(pad)
    pad_src = jnp.asarray((pad_ar * 997) % _N, jnp.int32)
    pad_dst = jnp.asarray(_N + pad_ar % (_ACC_ROWS - _N), jnp.int32)
    src2d = jnp.concatenate([src, pad_src]).reshape(_IDX_ROWS, _CHUNK)
    dst2d = jnp.concatenate([dst, pad_dst]).reshape(_IDX_ROWS, _CHUNK)
    zblk = jnp.zeros((_ZROWS, _F), jnp.float32)
    batch3 = batch.astype(jnp.int32).reshape(_NBLK, 1, _B)

    layer_args = []
    for (eps, p0, p1) in params:
        layer_args.append((
            jnp.reshape(eps, (1, 1)).astype(jnp.float32),
            p0[0].T,
            jnp.transpose(p0[1] * p0[2][..., None], (2, 1, 0)),
            p1[0].T,
            jnp.transpose(p1[1] * p1[2][..., None], (2, 1, 0)),
        ))

    h = x
    pooled = None
    for li in range(3):
        eps11, bw1T, w1, bw2T, w2 = layer_args[li]
        agg = _sc_segment_sum(h, src2d, dst2d, zblk)
        if li < 2:
            h = _tc_mid(eps11, h, agg[0], agg[1], bw1T, w1, bw2T, w2)
        else:
            pooled = _tc_last(eps11, h, agg[0], agg[1], bw1T, w1, bw2T, w2,
                              batch3)
    return (pooled, 0)
